# baseline (speedup 1.0000x reference)
import functools

import jax
import jax.numpy as jnp
from jax import lax
from jax.experimental import pallas as pl
from jax.experimental.pallas import tpu as pltpu
from jax.experimental.pallas import tpu_sc as plsc

N = 10000
E = 320000
D = 128

NP = 10240
EP = 331776
PAD_E = EP - (E + N)
ROWS_T = NP // 16
CH = 128
NCH_PASS = EP // 16 // CH
NCH_DEG = EP // 16 // CH
DEG_SPLIT = 60
DST_PAD = NP - 8

_mesh = plsc.VectorSubcoreMesh(core_axis_name="c", subcore_axis_name="s")


@functools.partial(
    pl.kernel,
    out_type=(
        jax.ShapeDtypeStruct((NP, D), jnp.float32),
        jax.ShapeDtypeStruct((NP, 16), jnp.float32),
        jax.ShapeDtypeStruct((NP, 16), jnp.float32),
    ),
    mesh=_mesh,
    scratch_types=(
        pltpu.VMEM_SHARED((NP, 16), jnp.float32),
        pltpu.VMEM((CH,), jnp.int32),
        pltpu.VMEM((CH, D), jnp.float32),
        pltpu.VMEM((CH, 16), jnp.float32),
        pltpu.VMEM((CH, 16), jnp.float32),
        pltpu.SemaphoreType.DMA,
    ),
)
def _sc_prep(x, perm, dst, ones16, zeros16, xp_out, degA_out, degB_out,
             acc16, idx_v, msg_v, ones_v, z16_v, sem):
    cid = lax.axis_index("c")
    sid = lax.axis_index("s")

    pltpu.sync_copy(ones16, ones_v)
    pltpu.sync_copy(zeros16, z16_v)
    for i in range(ROWS_T // CH):
        pltpu.sync_copy(z16_v, acc16.at[pl.ds(sid * ROWS_T + i * CH, CH)])
    plsc.subcore_barrier()

    def chunk(i, c):
        b = sid * (EP // 16) + i * CH
        pltpu.sync_copy(dst.at[pl.ds(b, CH)], idx_v)
        pltpu.sync_copy(ones_v, acc16.at[idx_v], add=True)
        return c

    @pl.when(cid == 0)
    def _sc0():
        for i in range(ROWS_T // CH):
            b = sid * ROWS_T + i * CH
            pltpu.sync_copy(perm.at[pl.ds(b, CH)], idx_v)
            pltpu.async_copy(x.at[idx_v], msg_v, sem).wait()
            pltpu.sync_copy(msg_v, xp_out.at[pl.ds(b, CH)])
        lax.fori_loop(0, DEG_SPLIT, chunk, 0)

    @pl.when(cid == 1)
    def _sc1():
        lax.fori_loop(DEG_SPLIT, NCH_DEG, chunk, 0)

    plsc.subcore_barrier()
    rows = pl.ds(sid * ROWS_T, ROWS_T)

    @pl.when(cid == 0)
    def _outA():
        pltpu.sync_copy(acc16.at[rows], degA_out.at[rows])

    @pl.when(cid == 1)
    def _outB():
        pltpu.sync_copy(acc16.at[rows], degB_out.at[rows])


@functools.partial(
    pl.kernel,
    out_type=(
        jax.ShapeDtypeStruct((NP, D), jnp.float32),
        jax.ShapeDtypeStruct((NP, D), jnp.float32),
    ),
    mesh=_mesh,
    scratch_types=(
        pltpu.VMEM_SHARED((NP, D), jnp.float32),
        pltpu.VMEM((CH,), jnp.int32),
        pltpu.VMEM((CH,), jnp.int32),
        pltpu.VMEM((CH, D), jnp.float32),
        pltpu.VMEM((CH, D), jnp.float32),
        pltpu.SemaphoreType.DMA,
    ),
)
def _sc_propagate(featP, featN, src, dst, zeros, outP, outN,
                  acc, idx_v, dst_v, msg_v, zbuf, sem):
    cid = lax.axis_index("c")
    sid = lax.axis_index("s")

    def half(feat, out):
        pltpu.sync_copy(zeros, zbuf)
        for i in range(ROWS_T // CH):
            pltpu.sync_copy(zbuf, acc.at[pl.ds(sid * ROWS_T + i * CH, CH)])
        plsc.subcore_barrier()

        def chunk(i, c):
            b = sid * (EP // 16) + i * CH
            pltpu.sync_copy(src.at[pl.ds(b, CH)], idx_v)
            pltpu.sync_copy(dst.at[pl.ds(b, CH)], dst_v)
            pltpu.async_copy(feat.at[idx_v], msg_v, sem).wait()
            pltpu.sync_copy(msg_v, acc.at[dst_v], add=True)
            return c

        lax.fori_loop(0, NCH_PASS, chunk, 0)
        plsc.subcore_barrier()
        pltpu.sync_copy(acc.at[pl.ds(sid * ROWS_T, ROWS_T)],
                        out.at[pl.ds(sid * ROWS_T, ROWS_T)])

    @pl.when(cid == 0)
    def _pos():
        half(featP, outP)

    @pl.when(cid == 1)
    def _neg():
        half(featN, outN)


def _tc_layer1_body(x_ref, xp_ref, degA_ref, degB_ref, w1_ref,
                    hsP_ref, hsN_ref, dinv_ref):
    dv = lax.rsqrt(degA_ref[...][:N, 0:1] + degB_ref[...][:N, 0:1])
    w1 = w1_ref[...]
    hsP_ref[...] = jnp.dot(x_ref[...], w1, preferred_element_type=jnp.float32) * dv
    hsN_ref[...] = jnp.dot(xp_ref[...][:N], w1, preferred_element_type=jnp.float32) * dv
    dinv_ref[...] = dv


def _tc_layer2_body(aP_ref, aN_ref, dinv_ref, a1_ref, w2_ref, hsP_ref, hsN_ref):
    dv = dinv_ref[...]
    a1 = a1_ref[...]
    w2 = w2_ref[...]
    for a_ref, o_ref in ((aP_ref, hsP_ref), (aN_ref, hsN_ref)):
        t = a_ref[...][:N] * dv
        z = jnp.where(t > 0, t, a1[None, :] * t)
        o_ref[...] = jnp.dot(z, w2, preferred_element_type=jnp.float32) * dv


def _tc_finish_body(aP_ref, aN_ref, dinv_ref, a2_ref, wd_ref, out_ref):
    dv = dinv_ref[...]
    a2 = a2_ref[...]
    tP = aP_ref[...][:N] * dv
    pos = jnp.where(tP > 0, tP, a2[None, :] * tP)
    tN = aN_ref[...][:N] * dv
    neg = jnp.where(tN > 0, tN, a2[None, :] * tN)
    summary = jax.nn.sigmoid(jnp.mean(pos, axis=0))
    svec = jnp.dot(wd_ref[...], summary[:, None], preferred_element_type=jnp.float32)
    pos_logits = jnp.dot(pos, svec, preferred_element_type=jnp.float32)
    neg_logits = jnp.dot(neg, svec, preferred_element_type=jnp.float32)

    def softplus(v):
        return jnp.maximum(v, 0.0) + jnp.log1p(jnp.exp(-jnp.abs(v)))

    l1 = jnp.mean(softplus(-pos_logits))
    l2 = jnp.mean(softplus(neg_logits))
    out_ref[...] = jnp.reshape(l1 + l2, (1, 1))


_tc_layer1 = pl.pallas_call(
    _tc_layer1_body,
    out_shape=(
        jax.ShapeDtypeStruct((N, D), jnp.float32),
        jax.ShapeDtypeStruct((N, D), jnp.float32),
        jax.ShapeDtypeStruct((N, 1), jnp.float32),
    ),
)

_tc_layer2 = pl.pallas_call(
    _tc_layer2_body,
    out_shape=(
        jax.ShapeDtypeStruct((N, D), jnp.float32),
        jax.ShapeDtypeStruct((N, D), jnp.float32),
    ),
)

_tc_finish = pl.pallas_call(
    _tc_finish_body,
    out_shape=jax.ShapeDtypeStruct((1, 1), jnp.float32),
)


def kernel(x, edges_pos, edges_neg, W1, a1, W2, a2, Wd):
    del edges_neg
    loop = jnp.arange(N, dtype=jnp.int32)
    src = jnp.concatenate(
        [edges_pos[0].astype(jnp.int32), loop,
         jnp.zeros((PAD_E,), jnp.int32)])
    dst = jnp.concatenate(
        [edges_pos[1].astype(jnp.int32), loop,
         jnp.full((PAD_E,), DST_PAD, jnp.int32)])
    perm_n = jax.random.permutation(jax.random.key(42), N).astype(jnp.int32)
    perm = jnp.concatenate([perm_n, jnp.zeros((NP - N,), jnp.int32)])
    zeros = jnp.zeros((CH, D), jnp.float32)
    ones16 = jnp.ones((CH, 16), jnp.float32)
    zeros16 = jnp.zeros((CH, 16), jnp.float32)

    xp, degA, degB = _sc_prep(x, perm, dst, ones16, zeros16)
    hsP, hsN, dinv = _tc_layer1(x, xp, degA, degB, W1)
    accP, accN = _sc_propagate(hsP, hsN, src, dst, zeros)
    hs2P, hs2N = _tc_layer2(accP, accN, dinv, a1, W2)
    acc2P, acc2N = _sc_propagate(hs2P, hs2N, src, dst, zeros)
    out = _tc_finish(acc2P, acc2N, dinv, a2, Wd)
    return out[0, 0]


# speedup vs baseline: 1.0137x; 1.0137x over previous
import functools

import jax
import jax.numpy as jnp
from jax import lax
from jax.experimental import pallas as pl
from jax.experimental.pallas import tpu as pltpu
from jax.experimental.pallas import tpu_sc as plsc

N = 10000
E = 320000
D = 128

NP = 10240
EP = 331776
PAD_E = EP - (E + N)
ROWS_T = NP // 16
CH = 128
NCH_PASS = EP // 16 // CH
NCH_DEG = EP // 16 // CH
DST_PAD = NP - 8

_mesh = plsc.VectorSubcoreMesh(core_axis_name="c", subcore_axis_name="s")


@functools.partial(
    pl.kernel,
    out_type=(
        jax.ShapeDtypeStruct((NP, D), jnp.float32),
        jax.ShapeDtypeStruct((NP, 16), jnp.float32),
    ),
    mesh=_mesh,
    scratch_types=(
        pltpu.VMEM_SHARED((NP, 16), jnp.float32),
        pltpu.VMEM((CH,), jnp.int32),
        pltpu.VMEM((CH, D), jnp.float32),
        pltpu.VMEM((CH, 16), jnp.float32),
        pltpu.VMEM((CH, 16), jnp.float32),
        pltpu.SemaphoreType.DMA,
    ),
)
def _sc_prep(x, perm, dst, ones16, zeros16, xp_out, deg_out,
             acc16, idx_v, msg_v, ones_v, z16_v, sem):
    cid = lax.axis_index("c")
    sid = lax.axis_index("s")

    @pl.when(cid == 0)
    def _gather_perm():
        for i in range(ROWS_T // CH):
            b = sid * ROWS_T + i * CH
            pltpu.sync_copy(perm.at[pl.ds(b, CH)], idx_v)
            pltpu.async_copy(x.at[idx_v], msg_v, sem).wait()
            pltpu.sync_copy(msg_v, xp_out.at[pl.ds(b, CH)])

    @pl.when(cid == 1)
    def _deg():
        pltpu.sync_copy(ones16, ones_v)
        pltpu.sync_copy(zeros16, z16_v)
        for i in range(ROWS_T // CH):
            pltpu.sync_copy(z16_v, acc16.at[pl.ds(sid * ROWS_T + i * CH, CH)])
        plsc.subcore_barrier()

        def chunk(i, c):
            b = sid * (EP // 16) + i * CH
            pltpu.sync_copy(dst.at[pl.ds(b, CH)], idx_v)
            pltpu.sync_copy(ones_v, acc16.at[idx_v], add=True)
            return c

        lax.fori_loop(0, NCH_DEG, chunk, 0)
        plsc.subcore_barrier()
        pltpu.sync_copy(acc16.at[pl.ds(sid * ROWS_T, ROWS_T)],
                        deg_out.at[pl.ds(sid * ROWS_T, ROWS_T)])


@functools.partial(
    pl.kernel,
    out_type=(
        jax.ShapeDtypeStruct((NP, D), jnp.float32),
        jax.ShapeDtypeStruct((NP, D), jnp.float32),
    ),
    mesh=_mesh,
    scratch_types=(
        pltpu.VMEM_SHARED((NP, D), jnp.float32),
        pltpu.VMEM((CH,), jnp.int32),
        pltpu.VMEM((CH,), jnp.int32),
        pltpu.VMEM((CH, D), jnp.float32),
        pltpu.VMEM((CH, D), jnp.float32),
        pltpu.SemaphoreType.DMA,
    ),
)
def _sc_propagate(featP, featN, src, dst, zeros, outP, outN,
                  acc, idx_v, dst_v, msg_v, zbuf, sem):
    cid = lax.axis_index("c")
    sid = lax.axis_index("s")

    def half(feat, out):
        pltpu.sync_copy(zeros, zbuf)
        for i in range(ROWS_T // CH):
            pltpu.sync_copy(zbuf, acc.at[pl.ds(sid * ROWS_T + i * CH, CH)])
        plsc.subcore_barrier()

        def chunk(i, c):
            b = sid * (EP // 16) + i * CH
            pltpu.sync_copy(src.at[pl.ds(b, CH)], idx_v)
            pltpu.sync_copy(dst.at[pl.ds(b, CH)], dst_v)
            pltpu.async_copy(feat.at[idx_v], msg_v, sem).wait()
            pltpu.sync_copy(msg_v, acc.at[dst_v], add=True)
            return c

        lax.fori_loop(0, NCH_PASS, chunk, 0)
        plsc.subcore_barrier()
        pltpu.sync_copy(acc.at[pl.ds(sid * ROWS_T, ROWS_T)],
                        out.at[pl.ds(sid * ROWS_T, ROWS_T)])

    @pl.when(cid == 0)
    def _pos():
        half(featP, outP)

    @pl.when(cid == 1)
    def _neg():
        half(featN, outN)


def _tc_layer1_body(x_ref, xp_ref, deg_ref, w1_ref, hsP_ref, hsN_ref, dinv_ref):
    dv = lax.rsqrt(deg_ref[...][:N, 0:1])
    w1 = w1_ref[...]
    hsP_ref[...] = jnp.dot(x_ref[...], w1, preferred_element_type=jnp.float32) * dv
    hsN_ref[...] = jnp.dot(xp_ref[...][:N], w1, preferred_element_type=jnp.float32) * dv
    dinv_ref[...] = dv


def _tc_layer2_body(aP_ref, aN_ref, dinv_ref, a1_ref, w2_ref, hsP_ref, hsN_ref):
    dv = dinv_ref[...]
    a1 = a1_ref[...]
    w2 = w2_ref[...]
    for a_ref, o_ref in ((aP_ref, hsP_ref), (aN_ref, hsN_ref)):
        t = a_ref[...][:N] * dv
        z = jnp.where(t > 0, t, a1[None, :] * t)
        o_ref[...] = jnp.dot(z, w2, preferred_element_type=jnp.float32) * dv


def _tc_finish_body(aP_ref, aN_ref, dinv_ref, a2_ref, wd_ref, out_ref):
    dv = dinv_ref[...]
    a2 = a2_ref[...]
    tP = aP_ref[...][:N] * dv
    pos = jnp.where(tP > 0, tP, a2[None, :] * tP)
    tN = aN_ref[...][:N] * dv
    neg = jnp.where(tN > 0, tN, a2[None, :] * tN)
    summary = jax.nn.sigmoid(jnp.mean(pos, axis=0))
    svec = jnp.dot(wd_ref[...], summary[:, None], preferred_element_type=jnp.float32)
    pos_logits = jnp.dot(pos, svec, preferred_element_type=jnp.float32)
    neg_logits = jnp.dot(neg, svec, preferred_element_type=jnp.float32)

    def softplus(v):
        return jnp.maximum(v, 0.0) + jnp.log1p(jnp.exp(-jnp.abs(v)))

    l1 = jnp.mean(softplus(-pos_logits))
    l2 = jnp.mean(softplus(neg_logits))
    out_ref[...] = jnp.reshape(l1 + l2, (1, 1))


_tc_layer1 = pl.pallas_call(
    _tc_layer1_body,
    out_shape=(
        jax.ShapeDtypeStruct((N, D), jnp.float32),
        jax.ShapeDtypeStruct((N, D), jnp.float32),
        jax.ShapeDtypeStruct((N, 1), jnp.float32),
    ),
)

_tc_layer2 = pl.pallas_call(
    _tc_layer2_body,
    out_shape=(
        jax.ShapeDtypeStruct((N, D), jnp.float32),
        jax.ShapeDtypeStruct((N, D), jnp.float32),
    ),
)

_tc_finish = pl.pallas_call(
    _tc_finish_body,
    out_shape=jax.ShapeDtypeStruct((1, 1), jnp.float32),
)


def kernel(x, edges_pos, edges_neg, W1, a1, W2, a2, Wd):
    del edges_neg
    loop = jnp.arange(N, dtype=jnp.int32)
    src = jnp.concatenate(
        [edges_pos[0].astype(jnp.int32), loop,
         jnp.zeros((PAD_E,), jnp.int32)])
    dst = jnp.concatenate(
        [edges_pos[1].astype(jnp.int32), loop,
         jnp.full((PAD_E,), DST_PAD, jnp.int32)])
    perm_n = jax.random.permutation(jax.random.key(42), N).astype(jnp.int32)
    perm = jnp.concatenate([perm_n, jnp.zeros((NP - N,), jnp.int32)])
    zeros = jnp.zeros((CH, D), jnp.float32)
    ones16 = jnp.ones((CH, 16), jnp.float32)
    zeros16 = jnp.zeros((CH, 16), jnp.float32)

    xp, deg = _sc_prep(x, perm, dst, ones16, zeros16)
    hsP, hsN, dinv = _tc_layer1(x, xp, deg, W1)
    accP, accN = _sc_propagate(hsP, hsN, src, dst, zeros)
    hs2P, hs2N = _tc_layer2(accP, accN, dinv, a1, W2)
    acc2P, acc2N = _sc_propagate(hs2P, hs2N, src, dst, zeros)
    out = _tc_finish(acc2P, acc2N, dinv, a2, Wd)
    return out[0, 0]
